# baseline (speedup 1.0000x reference)
import functools

import jax
import jax.numpy as jnp
from jax import lax
from jax.experimental import pallas as pl
from jax.experimental.pallas import tpu as pltpu
from jax.experimental.pallas import tpu_sc as plsc

B = 4096
F = 26
ROWS = 100001
D = 64

_INFO = plsc.get_sparse_core_info()
NC = _INFO.num_cores
NS = _INFO.num_subcores
L = _INFO.num_lanes
NW = NC * NS

G = F * D
PER_W = G // NW
RB = 4
UNROLL = 16

QCH = [25088, 25088, 25088, 24832]
QOFF = [0, 25088, 50176, 75264]
WIN = 100096
BUF = WIN + 128

_mesh = plsc.VectorSubcoreMesh(core_axis_name="c", subcore_axis_name="s")


@functools.partial(
    pl.kernel,
    mesh=_mesh,
    out_type=jax.ShapeDtypeStruct((G, B), jnp.float32),
    compiler_params=pltpu.CompilerParams(needs_layout_passes=False),
    scratch_types=[
        pltpu.VMEM((BUF,), jnp.float32),
        pltpu.VMEM((2, B), jnp.int32),
        pltpu.VMEM((RB, B), jnp.float32),
        pltpu.SemaphoreType.DMA,
    ],
)
def _emb_lookup(idx_hbm, tab_hbm, out_hbm, row_v, idx_v, res_v, insem):
    wid = lax.axis_index("s") * NC + lax.axis_index("c")
    gbase = wid * PER_W
    f0 = gbase // D
    pltpu.sync_copy(idx_hbm.at[f0], idx_v.at[0])
    pltpu.sync_copy(idx_hbm.at[jnp.minimum(f0 + 1, F - 1)], idx_v.at[1])

    def super_body(t, _):
        g0 = gbase + t * RB
        for u in range(RB):
            g = g0 + u
            s0 = g * ROWS
            a0 = (s0 // 128) * 128
            shift = s0 - a0
            cps = [
                pltpu.async_copy(
                    tab_hbm.at[pl.ds(a0 + QOFF[q], QCH[q])],
                    row_v.at[pl.ds(QOFF[q], QCH[q])],
                    insem,
                )
                for q in range(len(QCH))
            ]

            @pl.when(shift > WIN - ROWS)
            def _():
                pltpu.async_copy(
                    tab_hbm.at[pl.ds(a0 + WIN, 128)],
                    row_v.at[pl.ds(WIN, 128)],
                    insem,
                ).wait()

            for cp in cps:
                cp.wait()
            frel = g // D - f0

            def gather_body(i, _):
                for v in range(UNROLL):
                    sl = pl.ds((i * UNROLL + v) * L, L)
                    iv = idx_v[frel, sl] + shift
                    res_v[u, sl] = plsc.load_gather(row_v, [iv])
                return 0

            lax.fori_loop(0, B // (L * UNROLL), gather_body, 0)

        pltpu.sync_copy(res_v, out_hbm.at[pl.ds(g0, RB)])
        return 0

    lax.fori_loop(0, PER_W // RB, super_body, 0)


def kernel(indices, tables):
    idx_t = indices.astype(jnp.int32).T
    tab_flat = tables.transpose(0, 2, 1).reshape(G * ROWS)
    out_soa = _emb_lookup(idx_t, tab_flat)
    return out_soa.reshape(F, D, B).transpose(2, 0, 1)


# speedup vs baseline: 20.9008x; 20.9008x over previous
import functools

import jax
import jax.numpy as jnp
from jax import lax
from jax.experimental import pallas as pl
from jax.experimental.pallas import tpu as pltpu
from jax.experimental.pallas import tpu_sc as plsc

B = 4096
F = 26
ROWS = 100001
D = 64

_INFO = plsc.get_sparse_core_info()
NC = _INFO.num_cores
NS = _INFO.num_subcores
L = _INFO.num_lanes
NW = NC * NS

G = F * D
PER_W = G // NW
RB = 4
UNROLL = 16

_mesh = plsc.VectorSubcoreMesh(core_axis_name="c", subcore_axis_name="s")


@functools.partial(
    pl.kernel,
    mesh=_mesh,
    out_type=jax.ShapeDtypeStruct((G, B), jnp.float32),
    compiler_params=pltpu.CompilerParams(needs_layout_passes=False),
    scratch_types=[
        pltpu.VMEM((ROWS,), jnp.float32),
        pltpu.VMEM((2, B), jnp.int32),
        pltpu.VMEM((RB, B), jnp.float32),
    ],
)
def _emb_lookup(idx_hbm, tab_hbm, out_hbm, row_v, idx_v, res_v):
    wid = lax.axis_index("s") * NC + lax.axis_index("c")
    gbase = wid * PER_W
    f0 = gbase // D
    pltpu.sync_copy(idx_hbm.at[f0], idx_v.at[0])
    pltpu.sync_copy(idx_hbm.at[jnp.minimum(f0 + 1, F - 1)], idx_v.at[1])

    def super_body(t, _):
        g0 = gbase + t * RB
        for u in range(RB):
            g = g0 + u
            pltpu.sync_copy(tab_hbm.at[g], row_v)
            frel = g // D - f0

            def gather_body(i, _):
                for v in range(UNROLL):
                    sl = pl.ds((i * UNROLL + v) * L, L)
                    iv = idx_v[frel, sl]
                    res_v[u, sl] = plsc.load_gather(row_v, [iv])
                return 0

            lax.fori_loop(0, B // (L * UNROLL), gather_body, 0)

        pltpu.sync_copy(res_v, out_hbm.at[pl.ds(g0, RB)])
        return 0

    lax.fori_loop(0, PER_W // RB, super_body, 0)


def kernel(indices, tables):
    idx_t = indices.astype(jnp.int32).T
    tab_soa = tables.transpose(0, 2, 1).reshape(G, ROWS)
    out_soa = _emb_lookup(idx_t, tab_soa)
    return out_soa.reshape(F, D, B).transpose(2, 0, 1)
